# baseline (speedup 1.0000x reference)
import functools

import jax
import jax.numpy as jnp
import numpy as np
from jax import lax
from jax.experimental import pallas as pl
from jax.experimental.pallas import tpu as pltpu
from jax.experimental.pallas import tpu_sc as plsc

F32 = jnp.float32
I32 = jnp.int32


def _f(x):
    return np.float32(x)


def _erfc(u):
    one = _f(1.0)
    z = u * u
    p1 = z * _f(7.85386146e-05)
    p1 = p1 + _f(-0.000801019371)
    p1 = p1 * z
    p1 = p1 + _f(0.00518832775)
    p1 = p1 * z
    p1 = p1 + _f(-0.0268538129)
    p1 = p1 * z
    p1 = p1 + _f(0.112835854)
    p1 = p1 * z
    p1 = p1 + _f(-0.37612626)
    p1 = p1 * z
    p1 = p1 + _f(1.12837911)
    branch_small = one - u * p1
    nz = -z
    underflow = nz < _f(-88.7228394)
    e = jnp.exp(nz)
    au = jnp.abs(u)
    q = e * (one / au)
    lt2 = au < _f(2.0)
    w = one / z
    pa = w * _f(0.0232682)
    pa = pa + _f(-0.138703942)
    pa = pa * w
    pa = pa + _f(0.368742466)
    pa = pa * w
    pa = pa + _f(-0.582473278)
    pa = pa * w
    pa = pa + _f(0.621000469)
    pa = pa * w
    pa = pa + _f(-0.494451523)
    pa = pa * w
    pa = pa + _f(0.340488)
    pa = pa * w
    pa = pa + _f(-0.274112701)
    pa = pa * w
    pa = pa + _f(0.563825965)
    pb = w * _f(-10.477664)
    pb = pb + _f(12.9772)
    pb = pb * w
    pb = pb + _f(-7.49551868)
    pb = pb * w
    pb = pb + _f(2.92101908)
    pb = pb * w
    pb = pb + _f(-1.01526523)
    pb = pb * w
    pb = pb + _f(0.42184633)
    pb = pb * w
    pb = pb + _f(-0.282076746)
    pb = pb * w
    pb = pb + _f(0.564189494)
    psel = jnp.where(lt2, pa, pb)
    r = q * psel
    r = jnp.where(underflow, _f(0.0), r)
    refl = _f(2.0) - r
    r = jnp.where(u < _f(0.0), refl, r)
    return jnp.where(au < _f(1.0), branch_small, r)


def _mm1(x, w1t):
    h = lax.dot(x[:, :256], w1t[:256], preferred_element_type=F32)
    for k0 in range(256, x.shape[1], 256):
        h = h + lax.dot(x[:, k0:k0 + 256], w1t[k0:k0 + 256],
                        preferred_element_type=F32)
    return h


def _score_body(x_ref, w1t_ref, b1_ref, w2_ref, b2_ref, s_ref):
    x = x_ref[...]
    h = _mm1(x, w1t_ref[0]) + b1_ref[0]
    u = (-h) * _f(0.70710678118654752440084436210)
    g = (h * _f(0.5)) * _erfc(u)
    s = lax.dot(g, w2_ref[0], preferred_element_type=F32)
    s = s + b2_ref[0, 0, 0]
    s_ref[...] = s[:, 0:1][None]


def _scores(features, w1, b1, w2, b2):
    M, H = features.shape
    Hh = w1.shape[1]
    TM = 1024
    w1t = jnp.transpose(w1, (0, 2, 1))
    w2c = jnp.pad(w2[:, :, None], ((0, 0), (0, 0), (0, 127)))
    s = pl.pallas_call(
        _score_body,
        grid=(3, M // TM),
        in_specs=[
            pl.BlockSpec((TM, H), lambda h, m: (m, 0)),
            pl.BlockSpec((1, H, Hh), lambda h, m: (h, 0, 0)),
            pl.BlockSpec((1, 1, Hh), lambda h, m: (h, 0, 0)),
            pl.BlockSpec((1, Hh, 128), lambda h, m: (h, 0, 0)),
            pl.BlockSpec((1, 1, 1), lambda h, m: (h, 0, 0)),
        ],
        out_specs=pl.BlockSpec((1, TM, 1), lambda h, m: (h, m, 0)),
        out_shape=jax.ShapeDtypeStruct((3, M, 1), F32),
    )(features, w1t, b1[:, None, :], w2c, b2[:, None, None])
    return s[:, :, 0]


def _rank_body(sj_ref, si_ref, r_ref):
    si_all = si_ref[0]
    lane = lax.broadcasted_iota(I32, (128, 128), 1)
    sub = lax.broadcasted_iota(I32, (128, 128), 0)
    cols = []
    for c in range(32):
        si = si_all[:, c:c + 1]
        i_idx = sub * 32 + c

        def step(r, acc):
            sj = sj_ref[0, pl.ds(r, 1), :]
            j_idx = r * 128 + lane
            gt = sj > si
            tie = (sj == si) & (j_idx < i_idx)
            ind = jnp.where(gt | tie, _f(1.0), _f(0.0))
            return acc + jnp.sum(ind, axis=1, keepdims=True)

        acc = lax.fori_loop(0, 32, step, jnp.zeros((128, 1), F32))
        cols.append(acc.astype(I32))
    r_ref[0] = jnp.concatenate(cols, axis=1)


def _ranks(scores12):
    sj = scores12.reshape(12, 32, 128)
    si = scores12.reshape(12, 128, 32)
    r = pl.pallas_call(
        _rank_body,
        grid=(12,),
        in_specs=[
            pl.BlockSpec((1, 32, 128), lambda l: (l, 0, 0)),
            pl.BlockSpec((1, 128, 32), lambda l: (l, 0, 0)),
        ],
        out_specs=pl.BlockSpec((1, 128, 32), lambda l: (l, 0, 0)),
        out_shape=jax.ShapeDtypeStruct((12, 128, 32), I32),
    )(sj, si)
    return r.reshape(12, 4096)


_MESH = None


def _mesh():
    global _MESH
    if _MESH is None:
        _MESH = plsc.VectorSubcoreMesh(core_axis_name="c",
                                       subcore_axis_name="s")
    return _MESH


def _build_gidx(ranks, B, N):
    aux = np.zeros((12, 2, 16), np.int32)
    for L in range(12):
        aux[L, 0, :] = L * N
        aux[L, 1, :] = (L % 4) * N
    aux = jnp.asarray(aux)

    @functools.partial(
        pl.kernel,
        out_type=jax.ShapeDtypeStruct((12 * N,), I32),
        mesh=_mesh(),
        scratch_types=[
            pltpu.VMEM((N,), I32),
            pltpu.VMEM((N,), I32),
            pltpu.VMEM((N,), I32),
            pltpu.VMEM((2, 16), I32),
            pltpu.SemaphoreType.DMA,
        ],
    )
    def phase1(ranks_hbm, aux_hbm, gidx_hbm, rank_v, rg_v, vals_v, aux_v,
               sem):
        wid = lax.axis_index("s") * 2 + lax.axis_index("c")

        @pl.when(wid < 12)
        def _():
            pltpu.sync_copy(ranks_hbm.at[wid], rank_v)
            pltpu.sync_copy(aux_hbm.at[wid], aux_v)
            for c in range(N // 16):
                r16 = rank_v[pl.ds(c * 16, 16)]
                rg_v[pl.ds(c * 16, 16)] = r16 + aux_v[0]
                vals_v[pl.ds(c * 16, 16)] = (
                    aux_v[1] + (lax.iota(I32, 16) + (c * 16)))
            pltpu.async_copy(vals_v, gidx_hbm.at[rg_v], sem).wait()

    return phase1(ranks, aux)


def _gather_chunks(B, N):
    descs = []
    for b in range(B):
        for head in range(3):
            L = head * 4 + b
            k = 512 << head
            off = b * 3584 + (0, 512, 1536)[head]
            for r0 in range(0, k, 64):
                descs.append((L * N + r0, off + r0))
    return descs


def _gather_rows(flat, gidx, B, N):
    H = flat.shape[1]
    NB = B * 3584
    descs = _gather_chunks(B, N)
    per_w = len(descs) // 32

    @functools.partial(
        pl.kernel,
        out_type=jax.ShapeDtypeStruct((NB, H), F32),
        mesh=_mesh(),
        scratch_types=[
            pltpu.VMEM((64,), I32),
            pltpu.VMEM((64, H), F32),
            pltpu.SemaphoreType.DMA,
        ],
    )
    def phase2(gidx_hbm, flat_hbm, out_hbm, idx_v, rows_v, sem):
        wid = lax.axis_index("s") * 2 + lax.axis_index("c")
        for w in range(32):
            @pl.when(wid == w)
            def _(w=w):
                for src, dst in descs[w * per_w:(w + 1) * per_w]:
                    pltpu.sync_copy(gidx_hbm.at[pl.ds(src, 64)], idx_v)
                    pltpu.async_copy(flat_hbm.at[idx_v], rows_v, sem).wait()
                    pltpu.sync_copy(rows_v, out_hbm.at[pl.ds(dst, 64)])

    return phase2(gidx, flat)


def kernel(features, coords, gw1, gb1, gw2, gb2, lw1, lb1, lw2, lb2,
           dw1, db1, dw2, db2):
    del coords
    B, N, H = features.shape
    w1 = jnp.stack([gw1, lw1, dw1])
    b1 = jnp.stack([gb1, lb1, db1])
    w2 = jnp.stack([gw2[0], lw2[0], dw2[0]])
    b2 = jnp.stack([gb2[0], lb2[0], db2[0]])
    flat = features.reshape(B * N, H)
    scores = _scores(flat, w1, b1, w2, b2).reshape(3 * B, N)
    ranks = _ranks(scores)
    gidx = _build_gidx(ranks, B, N)
    rows = _gather_rows(flat, gidx, B, N)
    return rows.reshape(B, 3584, H)


# speedup vs baseline: 1.0229x; 1.0229x over previous
import functools

import jax
import jax.numpy as jnp
import numpy as np
from jax import lax
from jax.experimental import pallas as pl
from jax.experimental.pallas import tpu as pltpu
from jax.experimental.pallas import tpu_sc as plsc

F32 = jnp.float32
I32 = jnp.int32


def _f(x):
    return np.float32(x)


def _erfc(u):
    one = _f(1.0)
    z = u * u
    p1 = z * _f(7.85386146e-05)
    p1 = p1 + _f(-0.000801019371)
    p1 = p1 * z
    p1 = p1 + _f(0.00518832775)
    p1 = p1 * z
    p1 = p1 + _f(-0.0268538129)
    p1 = p1 * z
    p1 = p1 + _f(0.112835854)
    p1 = p1 * z
    p1 = p1 + _f(-0.37612626)
    p1 = p1 * z
    p1 = p1 + _f(1.12837911)
    branch_small = one - u * p1
    nz = -z
    underflow = nz < _f(-88.7228394)
    e = jnp.exp(nz)
    au = jnp.abs(u)
    q = e * (one / au)
    lt2 = au < _f(2.0)
    w = one / z
    pa = w * _f(0.0232682)
    pa = pa + _f(-0.138703942)
    pa = pa * w
    pa = pa + _f(0.368742466)
    pa = pa * w
    pa = pa + _f(-0.582473278)
    pa = pa * w
    pa = pa + _f(0.621000469)
    pa = pa * w
    pa = pa + _f(-0.494451523)
    pa = pa * w
    pa = pa + _f(0.340488)
    pa = pa * w
    pa = pa + _f(-0.274112701)
    pa = pa * w
    pa = pa + _f(0.563825965)
    pb = w * _f(-10.477664)
    pb = pb + _f(12.9772)
    pb = pb * w
    pb = pb + _f(-7.49551868)
    pb = pb * w
    pb = pb + _f(2.92101908)
    pb = pb * w
    pb = pb + _f(-1.01526523)
    pb = pb * w
    pb = pb + _f(0.42184633)
    pb = pb * w
    pb = pb + _f(-0.282076746)
    pb = pb * w
    pb = pb + _f(0.564189494)
    psel = jnp.where(lt2, pa, pb)
    r = q * psel
    r = jnp.where(underflow, _f(0.0), r)
    refl = _f(2.0) - r
    r = jnp.where(u < _f(0.0), refl, r)
    return jnp.where(au < _f(1.0), branch_small, r)


def _mm1(x, w1t):
    h = lax.dot(x[:, :256], w1t[:256], preferred_element_type=F32)
    for k0 in range(256, x.shape[1], 256):
        h = h + lax.dot(x[:, k0:k0 + 256], w1t[k0:k0 + 256],
                        preferred_element_type=F32)
    return h


def _score_body(x_ref, w1t_ref, b1_ref, w2_ref, b2_ref, s_ref):
    x = x_ref[...]
    h = _mm1(x, w1t_ref[0]) + b1_ref[0]
    u = (-h) * _f(0.70710678118654752440084436210)
    g = (h * _f(0.5)) * _erfc(u)
    s = lax.dot(g, w2_ref[0], preferred_element_type=F32)
    s = s + b2_ref[0, 0, 0]
    s_ref[...] = s[:, 0:1][None]


def _scores(features, w1, b1, w2, b2):
    M, H = features.shape
    Hh = w1.shape[1]
    TM = 1024
    w1t = jnp.transpose(w1, (0, 2, 1))
    w2c = jnp.pad(w2[:, :, None], ((0, 0), (0, 0), (0, 127)))
    s = pl.pallas_call(
        _score_body,
        grid=(3, M // TM),
        in_specs=[
            pl.BlockSpec((TM, H), lambda h, m: (m, 0)),
            pl.BlockSpec((1, H, Hh), lambda h, m: (h, 0, 0)),
            pl.BlockSpec((1, 1, Hh), lambda h, m: (h, 0, 0)),
            pl.BlockSpec((1, Hh, 128), lambda h, m: (h, 0, 0)),
            pl.BlockSpec((1, 1, 1), lambda h, m: (h, 0, 0)),
        ],
        out_specs=pl.BlockSpec((1, TM, 1), lambda h, m: (h, m, 0)),
        out_shape=jax.ShapeDtypeStruct((3, M, 1), F32),
    )(features, w1t, b1[:, None, :], w2c, b2[:, None, None])
    return s[:, :, 0]


def _rank_body(sj_ref, siT_ref, r_ref):
    siT = siT_ref[0]
    lane = lax.broadcasted_iota(I32, (128, 128), 1)
    sub = lax.broadcasted_iota(I32, (128, 128), 0)
    cols = []
    for c in range(32):
        si = siT[:, c:c + 1]

        def step_lo(r, acc):
            sj = sj_ref[0, pl.ds(r, 1), :]
            ind = jnp.where(sj >= si, _f(1.0), _f(0.0))
            return acc + jnp.sum(ind, axis=1, keepdims=True)

        def step_hi(r, acc):
            sj = sj_ref[0, pl.ds(r, 1), :]
            ind = jnp.where(sj > si, _f(1.0), _f(0.0))
            return acc + jnp.sum(ind, axis=1, keepdims=True)

        acc = lax.fori_loop(0, c, step_lo, jnp.zeros((128, 1), F32))
        acc = lax.fori_loop(c + 1, 32, step_hi, acc)
        sj = sj_ref[0, c:c + 1, :]
        tie = (sj == si) & (lane < sub)
        ind = jnp.where((sj > si) | tie, _f(1.0), _f(0.0))
        acc = acc + jnp.sum(ind, axis=1, keepdims=True)
        cols.append(acc.astype(I32))
    r_ref[0] = jnp.concatenate(cols, axis=1)


def _ranks(scores12):
    sj = scores12.reshape(12, 32, 128)
    siT = jnp.swapaxes(sj, 1, 2)
    r = pl.pallas_call(
        _rank_body,
        grid=(12,),
        in_specs=[
            pl.BlockSpec((1, 32, 128), lambda l: (l, 0, 0)),
            pl.BlockSpec((1, 128, 32), lambda l: (l, 0, 0)),
        ],
        out_specs=pl.BlockSpec((1, 128, 32), lambda l: (l, 0, 0)),
        out_shape=jax.ShapeDtypeStruct((12, 128, 32), I32),
    )(sj, siT)
    return jnp.swapaxes(r, 1, 2).reshape(12, 4096)


_MESH = None


def _mesh():
    global _MESH
    if _MESH is None:
        _MESH = plsc.VectorSubcoreMesh(core_axis_name="c",
                                       subcore_axis_name="s")
    return _MESH


def _build_gidx(ranks, B, N):
    aux = np.zeros((12, 2, 16), np.int32)
    for L in range(12):
        aux[L, 0, :] = L * N
        aux[L, 1, :] = (L % 4) * N
    aux = jnp.asarray(aux)

    @functools.partial(
        pl.kernel,
        out_type=jax.ShapeDtypeStruct((12 * N,), I32),
        mesh=_mesh(),
        scratch_types=[
            pltpu.VMEM((N,), I32),
            pltpu.VMEM((N,), I32),
            pltpu.VMEM((N,), I32),
            pltpu.VMEM((2, 16), I32),
            pltpu.SemaphoreType.DMA,
        ],
    )
    def phase1(ranks_hbm, aux_hbm, gidx_hbm, rank_v, rg_v, vals_v, aux_v,
               sem):
        wid = lax.axis_index("s") * 2 + lax.axis_index("c")

        @pl.when(wid < 12)
        def _():
            pltpu.sync_copy(ranks_hbm.at[wid], rank_v)
            pltpu.sync_copy(aux_hbm.at[wid], aux_v)
            for c in range(N // 16):
                r16 = rank_v[pl.ds(c * 16, 16)]
                rg_v[pl.ds(c * 16, 16)] = r16 + aux_v[0]
                vals_v[pl.ds(c * 16, 16)] = (
                    aux_v[1] + (lax.iota(I32, 16) + (c * 16)))
            pltpu.async_copy(vals_v, gidx_hbm.at[rg_v], sem).wait()

    return phase1(ranks, aux)


def _gather_chunks(B, N):
    descs = []
    for b in range(B):
        for head in range(3):
            L = head * 4 + b
            k = 512 << head
            off = b * 3584 + (0, 512, 1536)[head]
            for r0 in range(0, k, 64):
                descs.append((L * N + r0, off + r0))
    return descs


def _gather_rows(flat, gidx, B, N):
    H = flat.shape[1]
    NB = B * 3584
    descs = _gather_chunks(B, N)
    per_w = len(descs) // 32

    @functools.partial(
        pl.kernel,
        out_type=jax.ShapeDtypeStruct((NB, H), F32),
        mesh=_mesh(),
        scratch_types=[
            pltpu.VMEM((64,), I32),
            pltpu.VMEM((64, H), F32),
            pltpu.SemaphoreType.DMA,
        ],
    )
    def phase2(gidx_hbm, flat_hbm, out_hbm, idx_v, rows_v, sem):
        wid = lax.axis_index("s") * 2 + lax.axis_index("c")
        for w in range(32):
            @pl.when(wid == w)
            def _(w=w):
                for src, dst in descs[w * per_w:(w + 1) * per_w]:
                    pltpu.sync_copy(gidx_hbm.at[pl.ds(src, 64)], idx_v)
                    pltpu.async_copy(flat_hbm.at[idx_v], rows_v, sem).wait()
                    pltpu.sync_copy(rows_v, out_hbm.at[pl.ds(dst, 64)])

    return phase2(gidx, flat)


def kernel(features, coords, gw1, gb1, gw2, gb2, lw1, lb1, lw2, lb2,
           dw1, db1, dw2, db2):
    del coords
    B, N, H = features.shape
    w1 = jnp.stack([gw1, lw1, dw1])
    b1 = jnp.stack([gb1, lb1, db1])
    w2 = jnp.stack([gw2[0], lw2[0], dw2[0]])
    b2 = jnp.stack([gb2[0], lb2[0], db2[0]])
    flat = features.reshape(B * N, H)
    scores = _scores(flat, w1, b1, w2, b2).reshape(3 * B, N)
    ranks = _ranks(scores)
    gidx = _build_gidx(ranks, B, N)
    rows = _gather_rows(flat, gidx, B, N)
    return rows.reshape(B, 3584, H)
